# baseline (speedup 1.0000x reference)
import functools

import jax
import jax.numpy as jnp
from jax import lax
from jax.experimental import pallas as pl
from jax.experimental.pallas import tpu as pltpu
from jax.experimental.pallas import tpu_sc as plsc

B = 16384
D = 128
LAMBD = 0.1


_NC = 2
_NS = 16
_NW = _NC * _NS
_BPW = B // _NW
_GCH = 128
_NG = _BPW // _GCH
_NT = 2
_NCHUNK = _NT * _NG
_RING = 6


@functools.cache
def _get_sc_gather():
    mesh = plsc.VectorSubcoreMesh(core_axis_name="c", subcore_axis_name="s")

    @functools.partial(
        pl.kernel,
        mesh=mesh,
        out_type=[jax.ShapeDtypeStruct((B, D), jnp.float32) for _ in range(_NT)],
        scratch_types=[
            pltpu.VMEM((_NG, _GCH), jnp.int32),
            pltpu.VMEM((_NG, _GCH), jnp.int32),
            pltpu.VMEM((_RING, _GCH, D), jnp.float32),
        ] + [pltpu.SemaphoreType.DMA] * _RING,
    )
    def _sc_gather(tbl0_hbm, tbl1_hbm, idx0_hbm, idx1_hbm,
                   out0_hbm, out1_hbm, idx0_v, idx1_v, ring_v, *sems):
        wid = lax.axis_index("s") * _NC + lax.axis_index("c")
        out0 = wid * _BPW
        pltpu.sync_copy(idx0_hbm.at[wid], idx0_v)
        pltpu.sync_copy(idx1_hbm.at[wid], idx1_v)

        tbls = (tbl0_hbm, tbl1_hbm)
        idxs = (idx0_v, idx1_v)
        outs = (out0_hbm, out1_hbm)

        def fire_gather(k):
            t, j, s = k // _NG, k % _NG, k % _RING
            return pltpu.async_copy(tbls[t].at[idxs[t].at[j]], ring_v.at[s],
                                    sems[s])

        def fire_copyout(k):
            t, j, s = k // _NG, k % _NG, k % _RING
            return pltpu.async_copy(
                ring_v.at[s], outs[t].at[pl.ds(out0 + j * _GCH, _GCH)], sems[s])

        gathers = [None] * _NCHUNK
        tail = [None] * _RING
        for k in range(min(_RING, _NCHUNK)):
            gathers[k] = fire_gather(k)
        for k in range(_NCHUNK):
            gathers[k].wait()
            cp = fire_copyout(k)
            if k + _RING < _NCHUNK:
                cp.wait()
                gathers[k + _RING] = fire_gather(k + _RING)
            else:
                tail[k % _RING] = cp
        for cp in tail:
            if cp is not None:
                cp.wait()

    return _sc_gather


_RPG = 128
_G = B // _RPG
_GPC = 16
_NCH = _G // _GPC


def _tc_reg_body(sf, of, p_out, cs_out, co_out):
    i = pl.program_id(0)

    @pl.when(i == 0)
    def _init():
        cs_out[...] = jnp.zeros_like(cs_out)
        co_out[...] = jnp.zeros_like(co_out)

    sfb = sf[...]
    ofb = of[...]
    cs_out[...] += jnp.sum(sfb * sfb, axis=(0, 1))[None, :]
    co_out[...] += jnp.sum(ofb * ofb, axis=(0, 1))[None, :]
    p_out[...] = sfb * ofb


_tc_reg = pl.pallas_call(
    _tc_reg_body,
    grid=(_NCH,),
    in_specs=[
        pl.BlockSpec((_GPC, _RPG, D), lambda i: (i, 0, 0)),
        pl.BlockSpec((_GPC, _RPG, D), lambda i: (i, 0, 0)),
    ],
    out_specs=[
        pl.BlockSpec((_GPC, _RPG, D), lambda i: (i, 0, 0)),
        pl.BlockSpec((1, D), lambda i: (0, 0)),
        pl.BlockSpec((1, D), lambda i: (0, 0)),
    ],
    out_shape=[
        jax.ShapeDtypeStruct((_G, _RPG, D), jnp.float32),
        jax.ShapeDtypeStruct((1, D), jnp.float32),
        jax.ShapeDtypeStruct((1, D), jnp.float32),
    ],
    compiler_params=pltpu.CompilerParams(
        dimension_semantics=("arbitrary",),
    ),
)


def _tc_main_body(nf, ff, tg, p_in, cs, co, out, acc_m, acc_r):
    i = pl.program_id(0)

    @pl.when(i == 0)
    def _init():
        acc_m[0, 0] = 0.0
        acc_r[0, 0] = 0.0

    nfb = nf[...]
    ffb = ff[...]
    t = tg[...]

    un = jnp.sum(nfb * nfb, axis=2)
    vn = jnp.sum(ffb * ffb, axis=2)
    uv = jnp.sum(nfb * ffb, axis=2)
    s = uv * lax.rsqrt(un * vn)
    acc_m[0, 0] += jnp.sum(t * s - jnp.log(1.0 + jnp.exp(s)))

    c = lax.rsqrt(cs[...] * co[...])
    rs = jnp.sum(p_in[...] * c[None], axis=2)
    acc_r[0, 0] += jnp.sum(rs - jnp.log(1.0 + jnp.exp(rs)))

    @pl.when(i == _NCH - 1)
    def _fin():
        out[0, 0] = -(acc_m[0, 0] / B) - LAMBD * (acc_r[0, 0] / B)


_tc_main = pl.pallas_call(
    _tc_main_body,
    grid=(_NCH,),
    in_specs=[
        pl.BlockSpec((_GPC, _RPG, D), lambda i: (i, 0, 0)),
        pl.BlockSpec((_GPC, _RPG, D), lambda i: (i, 0, 0)),
        pl.BlockSpec((_GPC, _RPG), lambda i: (i, 0)),
        pl.BlockSpec((_GPC, _RPG, D), lambda i: (i, 0, 0)),
        pl.BlockSpec((1, D), lambda i: (0, 0)),
        pl.BlockSpec((1, D), lambda i: (0, 0)),
    ],
    out_specs=pl.BlockSpec(memory_space=pltpu.SMEM),
    out_shape=jax.ShapeDtypeStruct((1, 1), jnp.float32),
    scratch_shapes=[
        pltpu.SMEM((1, 1), jnp.float32),
        pltpu.SMEM((1, 1), jnp.float32),
    ],
    compiler_params=pltpu.CompilerParams(
        dimension_semantics=("arbitrary",),
    ),
)


def kernel(sources, contexts, targets, personas, pure_sources,
           node_embedding, node_noise_embedding, base_node_embedding):
    src = sources.astype(jnp.int32).reshape(_NW, _NG, _GCH)
    ctx = contexts.astype(jnp.int32).reshape(_NW, _NG, _GCH)
    pure = pure_sources.astype(jnp.int32).reshape(_NW, _NG, _GCH)
    pers = personas.astype(jnp.int32).reshape(_NW, _NG, _GCH)
    gather = _get_sc_gather()
    sf, of = gather(node_embedding, base_node_embedding, pure, pers)
    nf, ff = gather(node_embedding, node_noise_embedding, src, ctx)
    p, cs, co = _tc_reg(sf.reshape(_G, _RPG, D), of.reshape(_G, _RPG, D))
    out = _tc_main(nf.reshape(_G, _RPG, D), ff.reshape(_G, _RPG, D),
                   targets.reshape(_G, _RPG), p, cs, co)
    return out.reshape(())


# speedup vs baseline: 1.1815x; 1.1815x over previous
import functools

import jax
import jax.numpy as jnp
from jax import lax
from jax.experimental import pallas as pl
from jax.experimental.pallas import tpu as pltpu
from jax.experimental.pallas import tpu_sc as plsc

B = 16384
D = 128
LAMBD = 0.1

_NC = 2
_NS = 16
_NW = _NC * _NS
_BPW = B // _NW
_GCH = 128
_NG = _BPW // _GCH
_LC = D // 16
_NCHUNK = 2 * _NG
_NGRP = _BPW // 16


@functools.cache
def _get_sc_compute():
    mesh = plsc.VectorSubcoreMesh(core_axis_name="c", subcore_axis_name="s")

    @functools.partial(
        pl.kernel,
        mesh=mesh,
        out_type=[
            jax.ShapeDtypeStruct((_NW * _NGRP, 16), jnp.float32),
            jax.ShapeDtypeStruct((_NW * _NGRP, 16), jnp.float32),
            jax.ShapeDtypeStruct((_NW * _NGRP, 16), jnp.float32),
            jax.ShapeDtypeStruct((B, D), jnp.float32),
            jax.ShapeDtypeStruct((_NW, D), jnp.float32),
            jax.ShapeDtypeStruct((_NW, D), jnp.float32),
        ],
        scratch_types=[
            pltpu.VMEM((4, _NG, _GCH), jnp.int32),
            pltpu.VMEM((2, _GCH, D), jnp.float32),
            pltpu.VMEM((2, _GCH, D), jnp.float32),
            pltpu.VMEM((_NGRP, 16), jnp.float32),
            pltpu.VMEM((_NGRP, 16), jnp.float32),
            pltpu.VMEM((_NGRP, 16), jnp.float32),
            pltpu.VMEM((1, D), jnp.float32),
            pltpu.VMEM((1, D), jnp.float32),
            pltpu.SemaphoreType.DMA,
            pltpu.SemaphoreType.DMA,
            pltpu.SemaphoreType.DMA,
        ],
    )
    def _sc_compute(node_hbm, noise_hbm, base_hbm,
                    src_hbm, ctx_hbm, pure_hbm, pers_hbm,
                    uvs_hbm, uus_hbm, vvs_hbm, p_hbm, css_hbm, cos_hbm,
                    idx_v, ubuf, vbuf, uvb, uub, vvb, cs_st, co_st,
                    sem0, sem1, sem2):
        wid = lax.axis_index("s") * _NC + lax.axis_index("c")
        out0 = wid * _BPW
        sems = (sem0, sem1)
        pltpu.sync_copy(src_hbm.at[wid], idx_v.at[0])
        pltpu.sync_copy(ctx_hbm.at[wid], idx_v.at[1])
        pltpu.sync_copy(pure_hbm.at[wid], idx_v.at[2])
        pltpu.sync_copy(pers_hbm.at[wid], idx_v.at[3])

        lane = lax.iota(jnp.int32, 16)
        perms = [((lane + sh) % 16)[:, None] for sh in (8, 4, 2, 1)]
        dnums = lax.GatherDimensionNumbers(
            offset_dims=(), collapsed_slice_dims=(0,), start_index_map=(0,))

        def lsum(v):
            for p in perms:
                v = v + lax.gather(v, p, dnums, slice_sizes=(1,),
                                   mode=lax.GatherScatterMode.PROMISE_IN_BOUNDS)
            return v

        def fire(k):
            s = k % 2
            if k < _NG:
                tu, iu, tv, iv = node_hbm, 2, base_hbm, 3
                j = k
            else:
                tu, iu, tv, iv = node_hbm, 0, noise_hbm, 1
                j = k - _NG
            gu = pltpu.async_copy(tu.at[idx_v.at[iu, j]], ubuf.at[s], sems[s])
            gv = pltpu.async_copy(tv.at[idx_v.at[iv, j]], vbuf.at[s], sems[s])
            return gu, gv

        def reg_rows(s, accs):
            @plsc.parallel_loop(0, _GCH, unroll=2, carry=accs)
            def body(r, acc):
                au, av = acc
                nu, nv = [], []
                for c in range(_LC):
                    sl = pl.ds(c * 16, 16)
                    u = ubuf[s, r, sl]
                    v = vbuf[s, r, sl]
                    ubuf[s, r, sl] = u * v
                    nu.append(au[c] + u * u)
                    nv.append(av[c] + v * v)
                return (tuple(nu), tuple(nv))
            return body

        def main_rows(s, j):
            @plsc.parallel_loop(0, _GCH // 16)
            def body(g):
                zero = jnp.zeros((16,), jnp.float32)
                auv, auu, avv = zero, zero, zero
                for m in range(16):
                    r = g * 16 + m
                    sl0 = pl.ds(0, 16)
                    u = ubuf[s, r, sl0]
                    v = vbuf[s, r, sl0]
                    uv, uu, vv = u * v, u * u, v * v
                    for c in range(1, _LC):
                        sl = pl.ds(c * 16, 16)
                        u = ubuf[s, r, sl]
                        v = vbuf[s, r, sl]
                        uv = uv + u * v
                        uu = uu + u * u
                        vv = vv + v * v
                    msk = lane == m
                    auv = jnp.where(msk, lsum(uv), auv)
                    auu = jnp.where(msk, lsum(uu), auu)
                    avv = jnp.where(msk, lsum(vv), avv)
                grp = j * (_GCH // 16) + g
                uvb[grp] = auv
                uub[grp] = auu
                vvb[grp] = avv

        zero = jnp.zeros((16,), jnp.float32)
        accs = (tuple(zero for _ in range(_LC)), tuple(zero for _ in range(_LC)))

        gathers = [None] * _NCHUNK
        pend = [None, None]
        gathers[0] = fire(0)
        gathers[1] = fire(1)
        for k in range(_NCHUNK):
            s = k % 2
            gu, gv = gathers[k]
            gu.wait()
            gv.wait()
            if k < _NG:
                accs = reg_rows(s, accs)
                cp = pltpu.async_copy(
                    ubuf.at[s], p_hbm.at[pl.ds(out0 + k * _GCH, _GCH)], sems[s])
            else:
                main_rows(s, k - _NG)
                cp = None
            if k + 2 < _NCHUNK:
                if cp is not None:
                    cp.wait()
                gathers[k + 2] = fire(k + 2)
            else:
                pend[s] = cp
        for cp in pend:
            if cp is not None:
                cp.wait()

        au, av = accs
        for c in range(_LC):
            sl = pl.ds(c * 16, 16)
            cs_st[0, sl] = au[c]
            co_st[0, sl] = av[c]
        grp0 = wid * _NGRP
        cps = [
            pltpu.async_copy(uvb, uvs_hbm.at[pl.ds(grp0, _NGRP)], sem2),
            pltpu.async_copy(uub, uus_hbm.at[pl.ds(grp0, _NGRP)], sem2),
            pltpu.async_copy(vvb, vvs_hbm.at[pl.ds(grp0, _NGRP)], sem2),
            pltpu.async_copy(cs_st, css_hbm.at[pl.ds(wid, 1)], sem2),
            pltpu.async_copy(co_st, cos_hbm.at[pl.ds(wid, 1)], sem2),
        ]
        for cp in cps:
            cp.wait()

    return _sc_compute


_RPG = 128
_G = B // _RPG
_GPC = 32
_NCH = _G // _GPC


def _tc_loss_body(uvs, uus, vvs, p_in, css, cos, tg, out, acc_m, acc_r):
    i = pl.program_id(0)

    @pl.when(i == 0)
    def _init():
        acc_m[0, 0] = 0.0
        acc_r[0, 0] = 0.0

    s = uvs[...] * lax.rsqrt(uus[...] * vvs[...])
    t = tg[...]
    acc_m[0, 0] += jnp.sum(t * s - jnp.log(1.0 + jnp.exp(s)))

    cs = jnp.sum(css[...], axis=0, keepdims=True)
    co = jnp.sum(cos[...], axis=0, keepdims=True)
    c = lax.rsqrt(cs * co)
    rs = jnp.sum(p_in[...] * c[None], axis=2)
    acc_r[0, 0] += jnp.sum(rs - jnp.log(1.0 + jnp.exp(rs)))

    @pl.when(i == _NCH - 1)
    def _fin():
        out[0, 0] = -(acc_m[0, 0] / B) - LAMBD * (acc_r[0, 0] / B)


_tc_loss = pl.pallas_call(
    _tc_loss_body,
    grid=(_NCH,),
    in_specs=[
        pl.BlockSpec((_GPC, _RPG), lambda i: (i, 0)),
        pl.BlockSpec((_GPC, _RPG), lambda i: (i, 0)),
        pl.BlockSpec((_GPC, _RPG), lambda i: (i, 0)),
        pl.BlockSpec((_GPC, _RPG, D), lambda i: (i, 0, 0)),
        pl.BlockSpec((_NW, D), lambda i: (0, 0)),
        pl.BlockSpec((_NW, D), lambda i: (0, 0)),
        pl.BlockSpec((_GPC, _RPG), lambda i: (i, 0)),
    ],
    out_specs=pl.BlockSpec(memory_space=pltpu.SMEM),
    out_shape=jax.ShapeDtypeStruct((1, 1), jnp.float32),
    scratch_shapes=[
        pltpu.SMEM((1, 1), jnp.float32),
        pltpu.SMEM((1, 1), jnp.float32),
    ],
    compiler_params=pltpu.CompilerParams(
        dimension_semantics=("arbitrary",),
    ),
)


def kernel(sources, contexts, targets, personas, pure_sources,
           node_embedding, node_noise_embedding, base_node_embedding):
    src = sources.astype(jnp.int32).reshape(_NW, _NG, _GCH)
    ctx = contexts.astype(jnp.int32).reshape(_NW, _NG, _GCH)
    pure = pure_sources.astype(jnp.int32).reshape(_NW, _NG, _GCH)
    pers = personas.astype(jnp.int32).reshape(_NW, _NG, _GCH)
    uvs, uus, vvs, p, css, cos = _get_sc_compute()(
        node_embedding, node_noise_embedding, base_node_embedding,
        src, ctx, pure, pers)
    out = _tc_loss(uvs.reshape(_G, _RPG), uus.reshape(_G, _RPG),
                   vvs.reshape(_G, _RPG), p.reshape(_G, _RPG, D),
                   css, cos, targets.reshape(_G, _RPG))
    return out.reshape(())


# speedup vs baseline: 1.3567x; 1.1483x over previous
import functools

import jax
import jax.numpy as jnp
from jax import lax
from jax.experimental import pallas as pl
from jax.experimental.pallas import tpu as pltpu
from jax.experimental.pallas import tpu_sc as plsc

B = 16384
D = 128
LAMBD = 0.1

_NC = 2
_NS = 16
_NW = _NC * _NS
_BPW = B // _NW
_GCH = 128
_NG = _BPW // _GCH
_LC = D // 16
_NCHUNK = 2 * _NG
_NGRP = _BPW // 16


@functools.cache
def _get_sc_compute():
    mesh = plsc.VectorSubcoreMesh(core_axis_name="c", subcore_axis_name="s")

    @functools.partial(
        pl.kernel,
        mesh=mesh,
        out_type=[
            jax.ShapeDtypeStruct((_NW * _NG, D), jnp.float32),
            jax.ShapeDtypeStruct((_NW * _NG, D), jnp.float32),
            jax.ShapeDtypeStruct((_NW * _NG, D), jnp.float32),
            jax.ShapeDtypeStruct((B, D), jnp.float32),
            jax.ShapeDtypeStruct((_NW, D), jnp.float32),
            jax.ShapeDtypeStruct((_NW, D), jnp.float32),
        ],
        scratch_types=[
            pltpu.VMEM((4, _NG, _GCH), jnp.int32),
            pltpu.VMEM((3, _GCH, D), jnp.float32),
            pltpu.VMEM((3, _GCH, D), jnp.float32),
            pltpu.VMEM((_NG, D), jnp.float32),
            pltpu.VMEM((_NG, D), jnp.float32),
            pltpu.VMEM((_NG, D), jnp.float32),
            pltpu.VMEM((1, D), jnp.float32),
            pltpu.VMEM((1, D), jnp.float32),
            pltpu.SemaphoreType.DMA,
            pltpu.SemaphoreType.DMA,
            pltpu.SemaphoreType.DMA,
            pltpu.SemaphoreType.DMA,
        ],
    )
    def _sc_compute(node_hbm, noise_hbm, base_hbm,
                    src_hbm, ctx_hbm, pure_hbm, pers_hbm,
                    uvs_hbm, uus_hbm, vvs_hbm, p_hbm, css_hbm, cos_hbm,
                    idx_v, ubuf, vbuf, uvb, uub, vvb, cs_st, co_st,
                    sem0, sem1, sem2, sem3):
        wid = lax.axis_index("s") * _NC + lax.axis_index("c")
        out0 = wid * _BPW
        sems = (sem0, sem1, sem2)
        pltpu.sync_copy(src_hbm.at[wid], idx_v.at[0])
        pltpu.sync_copy(ctx_hbm.at[wid], idx_v.at[1])
        pltpu.sync_copy(pure_hbm.at[wid], idx_v.at[2])
        pltpu.sync_copy(pers_hbm.at[wid], idx_v.at[3])

        lane = lax.iota(jnp.int32, 16)
        perms = [((lane + sh) % 16)[:, None] for sh in (8, 4, 2, 1)]
        dnums = lax.GatherDimensionNumbers(
            offset_dims=(), collapsed_slice_dims=(0,), start_index_map=(0,))

        def lsum(v):
            for p in perms:
                v = v + lax.gather(v, p, dnums, slice_sizes=(1,),
                                   mode=lax.GatherScatterMode.PROMISE_IN_BOUNDS)
            return v

        def fire(k):
            s = k % 3
            if k < _NG:
                tu, iu, tv, iv = node_hbm, 2, base_hbm, 3
                j = k
            else:
                tu, iu, tv, iv = node_hbm, 0, noise_hbm, 1
                j = k - _NG
            gu = pltpu.async_copy(tu.at[idx_v.at[iu, j]], ubuf.at[s], sems[s])
            gv = pltpu.async_copy(tv.at[idx_v.at[iv, j]], vbuf.at[s], sems[s])
            return gu, gv

        def reg_rows(s, accs):
            @plsc.parallel_loop(0, _GCH, unroll=2, carry=accs)
            def body(r, acc):
                au, av = acc
                nu, nv = [], []
                for c in range(_LC):
                    sl = pl.ds(c * 16, 16)
                    u = ubuf[s, r, sl]
                    v = vbuf[s, r, sl]
                    ubuf[s, r, sl] = u * v
                    nu.append(au[c] + u * u)
                    nv.append(av[c] + v * v)
                return (tuple(nu), tuple(nv))
            return body

        def main_rows(s, j):
            @plsc.parallel_loop(0, _GCH // 16)
            def body(g):
                zero = jnp.zeros((16,), jnp.float32)
                auv, auu, avv = zero, zero, zero
                for m in range(16):
                    r = g * 16 + m
                    sl0 = pl.ds(0, 16)
                    u = ubuf[s, r, sl0]
                    v = vbuf[s, r, sl0]
                    uv, uu, vv = u * v, u * u, v * v
                    for c in range(1, _LC):
                        sl = pl.ds(c * 16, 16)
                        u = ubuf[s, r, sl]
                        v = vbuf[s, r, sl]
                        uv = uv + u * v
                        uu = uu + u * u
                        vv = vv + v * v
                    msk = lane == m
                    auv = jnp.where(msk, lsum(uv), auv)
                    auu = jnp.where(msk, lsum(uu), auu)
                    avv = jnp.where(msk, lsum(vv), avv)
                sl = pl.ds(g * 16, 16)
                uvb[j, sl] = auv
                uub[j, sl] = auu
                vvb[j, sl] = avv

        zero = jnp.zeros((16,), jnp.float32)
        accs = (tuple(zero for _ in range(_LC)), tuple(zero for _ in range(_LC)))

        gathers = [None] * _NCHUNK
        copyouts = [None] * _NCHUNK
        gathers[0] = fire(0)
        gathers[1] = fire(1)
        for k in range(_NCHUNK):
            s = k % 3
            gu, gv = gathers[k]
            gu.wait()
            gv.wait()
            if k + 2 < _NCHUNK:
                if k >= 1 and copyouts[k - 1] is not None:
                    copyouts[k - 1].wait()
                gathers[k + 2] = fire(k + 2)
            if k < _NG:
                accs = reg_rows(s, accs)
                copyouts[k] = pltpu.async_copy(
                    ubuf.at[s], p_hbm.at[pl.ds(out0 + k * _GCH, _GCH)], sems[s])
            else:
                main_rows(s, k - _NG)

        au, av = accs
        for c in range(_LC):
            sl = pl.ds(c * 16, 16)
            cs_st[0, sl] = au[c]
            co_st[0, sl] = av[c]
        row0 = wid * _NG
        cps = [
            pltpu.async_copy(uvb, uvs_hbm.at[pl.ds(row0, _NG)], sem3),
            pltpu.async_copy(uub, uus_hbm.at[pl.ds(row0, _NG)], sem3),
            pltpu.async_copy(vvb, vvs_hbm.at[pl.ds(row0, _NG)], sem3),
            pltpu.async_copy(cs_st, css_hbm.at[pl.ds(wid, 1)], sem3),
            pltpu.async_copy(co_st, cos_hbm.at[pl.ds(wid, 1)], sem3),
        ]
        for cp in cps:
            cp.wait()

    return _sc_compute


_RPG = 128
_G = B // _RPG
_GPC = 32
_NCH = _G // _GPC


def _tc_loss_body(uvs, uus, vvs, p_in, css, cos, tg, out, acc_m, acc_r):
    i = pl.program_id(0)

    @pl.when(i == 0)
    def _init():
        acc_m[0, 0] = 0.0
        acc_r[0, 0] = 0.0

    s = uvs[...] * lax.rsqrt(uus[...] * vvs[...])
    t = tg[...]
    acc_m[0, 0] += jnp.sum(t * s - jnp.log(1.0 + jnp.exp(s)))

    cs = jnp.sum(css[...], axis=0, keepdims=True)
    co = jnp.sum(cos[...], axis=0, keepdims=True)
    c = lax.rsqrt(cs * co)
    rs = jnp.sum(p_in[...] * c[None], axis=2)
    acc_r[0, 0] += jnp.sum(rs - jnp.log(1.0 + jnp.exp(rs)))

    @pl.when(i == _NCH - 1)
    def _fin():
        out[0, 0] = -(acc_m[0, 0] / B) - LAMBD * (acc_r[0, 0] / B)


_tc_loss = pl.pallas_call(
    _tc_loss_body,
    grid=(_NCH,),
    in_specs=[
        pl.BlockSpec((_GPC, _RPG), lambda i: (i, 0)),
        pl.BlockSpec((_GPC, _RPG), lambda i: (i, 0)),
        pl.BlockSpec((_GPC, _RPG), lambda i: (i, 0)),
        pl.BlockSpec((_GPC, _RPG, D), lambda i: (i, 0, 0)),
        pl.BlockSpec((_NW, D), lambda i: (0, 0)),
        pl.BlockSpec((_NW, D), lambda i: (0, 0)),
        pl.BlockSpec((_GPC, _RPG), lambda i: (i, 0)),
    ],
    out_specs=pl.BlockSpec(memory_space=pltpu.SMEM),
    out_shape=jax.ShapeDtypeStruct((1, 1), jnp.float32),
    scratch_shapes=[
        pltpu.SMEM((1, 1), jnp.float32),
        pltpu.SMEM((1, 1), jnp.float32),
    ],
    compiler_params=pltpu.CompilerParams(
        dimension_semantics=("arbitrary",),
    ),
)


def kernel(sources, contexts, targets, personas, pure_sources,
           node_embedding, node_noise_embedding, base_node_embedding):
    src = sources.astype(jnp.int32).reshape(_NW, _NG, _GCH)
    ctx = contexts.astype(jnp.int32).reshape(_NW, _NG, _GCH)
    pure = pure_sources.astype(jnp.int32).reshape(_NW, _NG, _GCH)
    pers = personas.astype(jnp.int32).reshape(_NW, _NG, _GCH)
    uvs, uus, vvs, p, css, cos = _get_sc_compute()(
        node_embedding, node_noise_embedding, base_node_embedding,
        src, ctx, pure, pers)
    out = _tc_loss(uvs, uus, vvs, p.reshape(_G, _RPG, D),
                   css, cos, targets.reshape(_G, _RPG))
    return out.reshape(())
